# Initial kernel scaffold; baseline (speedup 1.0000x reference)
import functools

import jax
import jax.numpy as jnp
from jax import lax
from jax.experimental import pallas as pl
from jax.experimental.pallas import tpu as pltpu
from jax.experimental.pallas import tpu_sc as plsc

_N = 100000
_E = 1600000
_G = 512
_D = 32
_H = 16
_NSUB = 16
_NCORE = 2

_IDX_W = 128
_CHUNK_ROWS = 16
_CHUNK_E = _CHUNK_ROWS * _IDX_W
_CHUNKS_PER_SUB = 49
_E_SUB = _CHUNK_E * _CHUNKS_PER_SUB
_E_PAD = _E_SUB * _NSUB
_ROWS_PER_SUB = _E_SUB // _IDX_W
_ROWS_TOTAL = _E_PAD // _IDX_W
_N_SUB = _N // _NSUB
_ACC_ROWS = _N + 16

_sc_mesh = plsc.VectorSubcoreMesh(
    core_axis_name="c", subcore_axis_name="s",
    num_cores=_NCORE, num_subcores=_NSUB)


@functools.partial(
    pl.kernel,
    out_type=jax.ShapeDtypeStruct((2 * _N, _H), jnp.float32),
    mesh=_sc_mesh,
    scratch_types=[
        pltpu.VMEM((_CHUNK_ROWS, _IDX_W), jnp.int32),
        pltpu.VMEM((_CHUNK_ROWS, _IDX_W), jnp.int32),
        pltpu.VMEM((_CHUNK_E, _H), jnp.float32),
        pltpu.VMEM_SHARED((_ACC_ROWS, _H), jnp.float32),
        pltpu.SemaphoreType.DMA,
        pltpu.SemaphoreType.DMA,
    ],
)
def _sc_agg(u_hbm, src_hbm, dst_hbm, z_hbm, srcv, dstv, rowsv, acc, gsem, ssem):
    c = lax.axis_index("c")
    s = lax.axis_index("s")

    pltpu.sync_copy(u_hbm.at[pl.ds(c * _N + s * _N_SUB, _N_SUB)],
                    acc.at[pl.ds(s * _N_SUB, _N_SUB)])
    plsc.subcore_barrier()

    @pl.loop(0, _CHUNKS_PER_SUB)
    def _chunk(k):
        base = s * _ROWS_PER_SUB + k * _CHUNK_ROWS
        pltpu.sync_copy(src_hbm.at[pl.ds(c * _ROWS_TOTAL + base, _CHUNK_ROWS)],
                        srcv)
        pltpu.sync_copy(dst_hbm.at[pl.ds(base, _CHUNK_ROWS)], dstv)
        gathers = [
            pltpu.async_copy(u_hbm.at[srcv.at[j]],
                             rowsv.at[pl.ds(j * _IDX_W, _IDX_W)], gsem)
            for j in range(_CHUNK_ROWS)
        ]
        for g in gathers:
            g.wait()
        scatters = [
            pltpu.async_copy(rowsv.at[pl.ds(j * _IDX_W, _IDX_W)],
                             acc.at[dstv.at[j]], ssem, add=True)
            for j in range(_CHUNK_ROWS)
        ]
        for sc in scatters:
            sc.wait()

    plsc.subcore_barrier()
    pltpu.sync_copy(acc.at[pl.ds(s * _N_SUB, _N_SUB)],
                    z_hbm.at[pl.ds(c * _N + s * _N_SUB, _N_SUB)])


_B_ENC = 10000


def _enc_body(x_ref, pe_ref, w0, b0, w1, b1, w0a, w0b, u_ref):
    h = jnp.maximum(
        jnp.dot(x_ref[...], w0[...], preferred_element_type=jnp.float32)
        + b0[...], 0.0)
    h = jnp.dot(h, w1[...], preferred_element_type=jnp.float32) + b1[...]
    u = (jnp.dot(h, w0a[...], preferred_element_type=jnp.float32)
         + jnp.dot(pe_ref[...], w0b[...], preferred_element_type=jnp.float32))
    u_ref[0, :, :] = u[:, :_H]
    u_ref[1, :, :] = u[:, _H:]


def _full(shape):
    return pl.BlockSpec(shape, lambda i: tuple(0 for _ in shape))


def _tc_encoder(x, pe, w0, b0, w1, b1, w0a, w0b):
    grid = (_N // _B_ENC,)
    return pl.pallas_call(
        _enc_body,
        grid=grid,
        in_specs=[
            pl.BlockSpec((_B_ENC, 37), lambda i: (i, 0)),
            pl.BlockSpec((_B_ENC, 20), lambda i: (i, 0)),
            _full((37, _D)), _full((1, _D)), _full((_D, _D)), _full((1, _D)),
            _full((_D, _D)), _full((20, _D)),
        ],
        out_specs=pl.BlockSpec((2, _B_ENC, _H), lambda i: (0, i, 0)),
        out_shape=jax.ShapeDtypeStruct((2, _N, _H), jnp.float32),
    )(x, pe, w0, b0, w1, b1, w0a, w0b)


def _mid_body(z_ref, b0, w1, b1, w0n, u_ref):
    z32 = jnp.concatenate([z_ref[0], z_ref[1]], axis=1)
    t = jnp.maximum(z32 + b0[...], 0.0)
    h = jnp.maximum(
        jnp.dot(t, w1[...], preferred_element_type=jnp.float32) + b1[...], 0.0)
    u = jnp.dot(h, w0n[...], preferred_element_type=jnp.float32)
    u_ref[0, :, :] = u[:, :_H]
    u_ref[1, :, :] = u[:, _H:]


def _tc_mid(z, b0, w1, b1, w0n):
    grid = (_N // _B_ENC,)
    return pl.pallas_call(
        _mid_body,
        grid=grid,
        in_specs=[
            pl.BlockSpec((2, _B_ENC, _H), lambda i: (0, i, 0)),
            _full((1, _D)), _full((_D, _D)), _full((1, _D)), _full((_D, _D)),
        ],
        out_specs=pl.BlockSpec((2, _B_ENC, _H), lambda i: (0, i, 0)),
        out_shape=jax.ShapeDtypeStruct((2, _N, _H), jnp.float32),
    )(z, b0, w1, b1, w0n)


_B_FIN = 2500
_NB_FIN = _N // _B_FIN


def _fin_body(z_ref, batch_ref, b0, w1, b1, ow0, ob0, ow1, ob1, out_ref, acc):
    i = pl.program_id(0)

    @pl.when(i == 0)
    def _():
        acc[...] = jnp.zeros_like(acc)

    z32 = jnp.concatenate([z_ref[0], z_ref[1]], axis=1)
    t = jnp.maximum(z32 + b0[...], 0.0)
    h = jnp.maximum(
        jnp.dot(t, w1[...], preferred_element_type=jnp.float32) + b1[...], 0.0)
    v = jnp.dot(h, ow0[...], preferred_element_type=jnp.float32)
    bvec = batch_ref[0, 0, :]
    onehot = (lax.broadcasted_iota(jnp.int32, (_G, _B_FIN), 0)
              == bvec[None, :]).astype(jnp.float32)
    acc[...] += jnp.dot(onehot, v, preferred_element_type=jnp.float32)

    @pl.when(i == _NB_FIN - 1)
    def _():
        p = jnp.maximum(acc[...] + ob0[...], 0.0)
        out_ref[...] = (jnp.dot(p, ow1[...], preferred_element_type=jnp.float32)
                        + ob1[...])


def _tc_final(z, batch3, b0, w1, b1, ow0, ob0, ow1, ob1):
    return pl.pallas_call(
        _fin_body,
        grid=(_NB_FIN,),
        in_specs=[
            pl.BlockSpec((2, _B_FIN, _H), lambda i: (0, i, 0)),
            pl.BlockSpec((1, 1, _B_FIN), lambda i: (i, 0, 0)),
            _full((1, _D)), _full((_D, _D)), _full((1, _D)), _full((_D, _D)),
            _full((1, _D)), _full((_D, 2)), _full((1, 2)),
        ],
        out_specs=pl.BlockSpec((_G, 2), lambda i: (0, 0)),
        out_shape=jax.ShapeDtypeStruct((_G, 2), jnp.float32),
        scratch_shapes=[pltpu.VMEM((_G, _D), jnp.float32)],
    )(z, batch3, b0, w1, b1, ow0, ob0, ow1, ob1)


def kernel(x, node_pe, edge_index, batch,
           enc_W0, enc_b0, enc_W1, enc_b1,
           gin0_W0, gin0_b0, gin0_W1, gin0_b1,
           gin1_W0, gin1_b0, gin1_W1, gin1_b1,
           gin2_W0, gin2_b0, gin2_W1, gin2_b1,
           out_W0, out_b0, out_W1, out_b1):
    src = edge_index[0]
    dst = edge_index[1]
    pad = _E_PAD - _E
    src_p = jnp.concatenate([src, jnp.zeros((pad,), jnp.int32)])
    dst_p = jnp.concatenate([dst, jnp.full((pad,), _N, jnp.int32)])
    src_idx = jnp.concatenate([src_p, src_p + _N]).reshape(2 * _ROWS_TOTAL,
                                                           _IDX_W)
    dst_idx = dst_p.reshape(_ROWS_TOTAL, _IDX_W)
    batch3 = batch.reshape(_NB_FIN, 1, _B_FIN)

    r = lambda b: b.reshape(1, -1)

    u = _tc_encoder(x, node_pe, enc_W0, r(enc_b0), enc_W1, r(enc_b1),
                    gin0_W0[:_D], gin0_W0[_D:])
    z = _sc_agg(u.reshape(2 * _N, _H), src_idx, dst_idx)
    u = _tc_mid(z.reshape(2, _N, _H), r(gin0_b0), gin0_W1, r(gin0_b1), gin1_W0)
    z = _sc_agg(u.reshape(2 * _N, _H), src_idx, dst_idx)
    u = _tc_mid(z.reshape(2, _N, _H), r(gin1_b0), gin1_W1, r(gin1_b1), gin2_W0)
    z = _sc_agg(u.reshape(2 * _N, _H), src_idx, dst_idx)
    return _tc_final(z.reshape(2, _N, _H), batch3, r(gin2_b0), gin2_W1,
                     r(gin2_b1), out_W0, r(out_b0), out_W1, r(out_b1))


# speedup vs baseline: 11.7204x; 11.7204x over previous
import functools

import jax
import jax.numpy as jnp
from jax import lax
from jax.experimental import pallas as pl
from jax.experimental.pallas import tpu as pltpu
from jax.experimental.pallas import tpu_sc as plsc

_N = 100000
_E = 1600000
_G = 512
_D = 32
_H = 16
_NSUB = 16
_NCORE = 2

_IDX_W = 128
_CHUNK_ROWS = 8
_CHUNK_E = _CHUNK_ROWS * _IDX_W
_CHUNKS_PER_SUB = 98
_E_SUB = _CHUNK_E * _CHUNKS_PER_SUB
_E_PAD = _E_SUB * _NSUB
_ROWS_PER_SUB = _E_SUB // _IDX_W
_ROWS_TOTAL = _E_PAD // _IDX_W
_N_PAD = 100096
_N_SUB = _N_PAD // _NSUB
_ACC_ROWS = _N_PAD

def _sc_agg_body(u_hbm, src_hbm, dst_hbm, z_hbm, srcv, dstv, rowsv, acc,
                 gsem, ssem):
    c = lax.axis_index("c")
    s = lax.axis_index("s")

    pltpu.sync_copy(u_hbm.at[pl.ds(c * _N_PAD + s * _N_SUB, _N_SUB)],
                    acc.at[pl.ds(s * _N_SUB, _N_SUB)])
    plsc.subcore_barrier()

    @pl.loop(0, _CHUNKS_PER_SUB)
    def _chunk(k):
        base = s * _ROWS_PER_SUB + k * _CHUNK_ROWS
        pltpu.sync_copy(src_hbm.at[pl.ds(c * _ROWS_TOTAL + base, _CHUNK_ROWS)],
                        srcv)
        pltpu.sync_copy(dst_hbm.at[pl.ds(base, _CHUNK_ROWS)], dstv)
        gathers = [
            pltpu.async_copy(u_hbm.at[srcv.at[j]],
                             rowsv.at[pl.ds(j * _IDX_W, _IDX_W)], gsem)
            for j in range(_CHUNK_ROWS)
        ]
        for g in gathers:
            g.wait()
        scatters = [
            pltpu.async_copy(rowsv.at[pl.ds(j * _IDX_W, _IDX_W)],
                             acc.at[dstv.at[j]], ssem, add=True)
            for j in range(_CHUNK_ROWS)
        ]
        for sc in scatters:
            sc.wait()

    plsc.subcore_barrier()
    pltpu.sync_copy(acc.at[pl.ds(s * _N_SUB, _N_SUB)],
                    z_hbm.at[pl.ds(c * _N_PAD + s * _N_SUB, _N_SUB)])


@functools.cache
def _build_sc_agg():
    mesh = plsc.VectorSubcoreMesh(
        core_axis_name="c", subcore_axis_name="s",
        num_cores=_NCORE, num_subcores=_NSUB)
    return pl.kernel(
        _sc_agg_body,
        out_type=jax.ShapeDtypeStruct((2 * _N_PAD, _H), jnp.float32),
        mesh=mesh,
        compiler_params=pltpu.CompilerParams(use_tc_tiling_on_sc=False),
        scratch_types=[
            pltpu.VMEM((_CHUNK_ROWS, _IDX_W), jnp.int32),
            pltpu.VMEM((_CHUNK_ROWS, _IDX_W), jnp.int32),
            pltpu.VMEM((_CHUNK_E, _H), jnp.float32),
            pltpu.VMEM_SHARED((_ACC_ROWS, _H), jnp.float32),
            pltpu.SemaphoreType.DMA,
            pltpu.SemaphoreType.DMA,
        ],
    )


_B_ENC = 10000


def _enc_body(x_ref, pe_ref, w0, b0, w1, b1, w0a, w0b, u_ref):
    h = jnp.maximum(
        jnp.dot(x_ref[...], w0[...], preferred_element_type=jnp.float32)
        + b0[...], 0.0)
    h = jnp.dot(h, w1[...], preferred_element_type=jnp.float32) + b1[...]
    u = (jnp.dot(h, w0a[...], preferred_element_type=jnp.float32)
         + jnp.dot(pe_ref[...], w0b[...], preferred_element_type=jnp.float32))
    u_ref[0, :, :] = u[:, :_H]
    u_ref[1, :, :] = u[:, _H:]


def _full(shape):
    return pl.BlockSpec(shape, lambda i: tuple(0 for _ in shape))


def _tc_encoder(x, pe, w0, b0, w1, b1, w0a, w0b):
    grid = (_N // _B_ENC,)
    return pl.pallas_call(
        _enc_body,
        grid=grid,
        in_specs=[
            pl.BlockSpec((_B_ENC, 37), lambda i: (i, 0)),
            pl.BlockSpec((_B_ENC, 20), lambda i: (i, 0)),
            _full((37, _D)), _full((1, _D)), _full((_D, _D)), _full((1, _D)),
            _full((_D, _D)), _full((20, _D)),
        ],
        out_specs=pl.BlockSpec((2, _B_ENC, _H), lambda i: (0, i, 0)),
        out_shape=jax.ShapeDtypeStruct((2, _N_PAD, _H), jnp.float32),
    )(x, pe, w0, b0, w1, b1, w0a, w0b)


def _mid_body(z_ref, b0, w1, b1, w0n, u_ref):
    z32 = jnp.concatenate([z_ref[0], z_ref[1]], axis=1)
    t = jnp.maximum(z32 + b0[...], 0.0)
    h = jnp.maximum(
        jnp.dot(t, w1[...], preferred_element_type=jnp.float32) + b1[...], 0.0)
    u = jnp.dot(h, w0n[...], preferred_element_type=jnp.float32)
    u_ref[0, :, :] = u[:, :_H]
    u_ref[1, :, :] = u[:, _H:]


def _tc_mid(z, b0, w1, b1, w0n):
    grid = (_N // _B_ENC,)
    return pl.pallas_call(
        _mid_body,
        grid=grid,
        in_specs=[
            pl.BlockSpec((2, _B_ENC, _H), lambda i: (0, i, 0)),
            _full((1, _D)), _full((_D, _D)), _full((1, _D)), _full((_D, _D)),
        ],
        out_specs=pl.BlockSpec((2, _B_ENC, _H), lambda i: (0, i, 0)),
        out_shape=jax.ShapeDtypeStruct((2, _N_PAD, _H), jnp.float32),
    )(z, b0, w1, b1, w0n)


_B_FIN = 2000
_NB_FIN = _N // _B_FIN


def _fin_body(z_ref, batch_ref, b0, w1, b1, ow0, ob0, ow1, ob1, out_ref, acc):
    i = pl.program_id(0)

    @pl.when(i == 0)
    def _():
        acc[...] = jnp.zeros_like(acc)

    z32 = jnp.concatenate([z_ref[0], z_ref[1]], axis=1)
    t = jnp.maximum(z32 + b0[...], 0.0)
    h = jnp.maximum(
        jnp.dot(t, w1[...], preferred_element_type=jnp.float32) + b1[...], 0.0)
    v = jnp.dot(h, ow0[...], preferred_element_type=jnp.float32)
    bvec = batch_ref[0, 0, :]
    onehot = (lax.broadcasted_iota(jnp.int32, (_G, _B_FIN), 0)
              == bvec[None, :]).astype(jnp.float32)
    acc[...] += jnp.dot(onehot, v, preferred_element_type=jnp.float32)

    @pl.when(i == _NB_FIN - 1)
    def _():
        p = jnp.maximum(acc[...] + ob0[...], 0.0)
        out_ref[...] = (jnp.dot(p, ow1[...], preferred_element_type=jnp.float32)
                        + ob1[...])


def _tc_final(z, batch3, b0, w1, b1, ow0, ob0, ow1, ob1):
    return pl.pallas_call(
        _fin_body,
        grid=(_NB_FIN,),
        in_specs=[
            pl.BlockSpec((2, _B_FIN, _H), lambda i: (0, i, 0)),
            pl.BlockSpec((1, 1, _B_FIN), lambda i: (i, 0, 0)),
            _full((1, _D)), _full((_D, _D)), _full((1, _D)), _full((_D, _D)),
            _full((1, _D)), _full((_D, 2)), _full((1, 2)),
        ],
        out_specs=pl.BlockSpec((_G, 2), lambda i: (0, 0)),
        out_shape=jax.ShapeDtypeStruct((_G, 2), jnp.float32),
        scratch_shapes=[pltpu.VMEM((_G, _D), jnp.float32)],
    )(z, batch3, b0, w1, b1, ow0, ob0, ow1, ob1)


def kernel(x, node_pe, edge_index, batch,
           enc_W0, enc_b0, enc_W1, enc_b1,
           gin0_W0, gin0_b0, gin0_W1, gin0_b1,
           gin1_W0, gin1_b0, gin1_W1, gin1_b1,
           gin2_W0, gin2_b0, gin2_W1, gin2_b1,
           out_W0, out_b0, out_W1, out_b1):
    src = edge_index[0]
    dst = edge_index[1]
    pad = _E_PAD - _E
    src_p = jnp.concatenate([src, jnp.zeros((pad,), jnp.int32)])
    dst_p = jnp.concatenate([dst, jnp.full((pad,), _N, jnp.int32)])
    src_idx = jnp.concatenate([src_p, src_p + _N_PAD]).reshape(2 * _ROWS_TOTAL,
                                                           _IDX_W)
    dst_idx = dst_p.reshape(_ROWS_TOTAL, _IDX_W)
    batch3 = batch.reshape(_NB_FIN, 1, _B_FIN)

    r = lambda b: b.reshape(1, -1)
    _sc_agg = _build_sc_agg()

    u = _tc_encoder(x, node_pe, enc_W0, r(enc_b0), enc_W1, r(enc_b1),
                    gin0_W0[:_D], gin0_W0[_D:])
    z = _sc_agg(u.reshape(2 * _N_PAD, _H), src_idx, dst_idx)
    u = _tc_mid(z.reshape(2, _N_PAD, _H), r(gin0_b0), gin0_W1, r(gin0_b1), gin1_W0)
    z = _sc_agg(u.reshape(2 * _N_PAD, _H), src_idx, dst_idx)
    u = _tc_mid(z.reshape(2, _N_PAD, _H), r(gin1_b0), gin1_W1, r(gin1_b1), gin2_W0)
    z = _sc_agg(u.reshape(2 * _N_PAD, _H), src_idx, dst_idx)
    return _tc_final(z.reshape(2, _N_PAD, _H), batch3, r(gin2_b0), gin2_W1,
                     r(gin2_b1), out_W0, r(out_b0), out_W1, r(out_b1))


# speedup vs baseline: 12.3155x; 1.0508x over previous
import functools

import jax
import jax.numpy as jnp
from jax import lax
from jax.experimental import pallas as pl
from jax.experimental.pallas import tpu as pltpu
from jax.experimental.pallas import tpu_sc as plsc

_N = 100000
_E = 1600000
_G = 512
_D = 32
_H = 16
_NSUB = 16
_NCORE = 2

_IDX_W = 128
_CHUNK_ROWS = 4
_CHUNK_E = _CHUNK_ROWS * _IDX_W
_CHUNKS_PER_SUB = 196
_E_SUB = _CHUNK_E * _CHUNKS_PER_SUB
_E_PAD = _E_SUB * _NSUB
_ROWS_PER_SUB = _E_SUB // _IDX_W
_ROWS_TOTAL = _E_PAD // _IDX_W
_N_PAD = 100096
_N_SUB = _N_PAD // _NSUB
_ACC_ROWS = _N_PAD

def _sc_agg_body(u_hbm, src_hbm, dst_hbm, z_hbm, srcv, dstv, rowsv, acc,
                 gsem0, gsem1, ssem0, ssem1):
    c = lax.axis_index("c")
    s = lax.axis_index("s")
    gsems = (gsem0, gsem1)
    ssems = (ssem0, ssem1)

    pltpu.sync_copy(u_hbm.at[pl.ds(c * _N_PAD + s * _N_SUB, _N_SUB)],
                    acc.at[pl.ds(s * _N_SUB, _N_SUB)])
    plsc.subcore_barrier()

    def load_idx(k, b):
        base = s * _ROWS_PER_SUB + k * _CHUNK_ROWS
        pltpu.sync_copy(src_hbm.at[pl.ds(c * _ROWS_TOTAL + base, _CHUNK_ROWS)],
                        srcv.at[b])
        pltpu.sync_copy(dst_hbm.at[pl.ds(base, _CHUNK_ROWS)], dstv.at[b])

    def issue_gathers(b):
        for j in range(_CHUNK_ROWS):
            pltpu.async_copy(u_hbm.at[srcv.at[b].at[j]],
                             rowsv.at[b].at[pl.ds(j * _IDX_W, _IDX_W)],
                             gsems[b])

    def issue_scatters(b):
        for j in range(_CHUNK_ROWS):
            pltpu.async_copy(rowsv.at[b].at[pl.ds(j * _IDX_W, _IDX_W)],
                             acc.at[dstv.at[b].at[j]], ssems[b], add=True)

    def drain(b, sem):
        pltpu.make_async_copy(u_hbm.at[pl.ds(0, _CHUNK_E)], rowsv.at[b],
                              sem).wait()

    for b in (0, 1):
        load_idx(b, b)
        issue_gathers(b)

    @pl.loop(0, _CHUNKS_PER_SUB // 2)
    def _step(t):
        for b in (0, 1):
            drain(b, gsems[b])
            issue_scatters(b)
        for b in (0, 1):
            @pl.when(t < _CHUNKS_PER_SUB // 2 - 1)
            def _():
                load_idx(2 * t + 2 + b, b)
            drain(b, ssems[b])

            @pl.when(t < _CHUNKS_PER_SUB // 2 - 1)
            def _():
                issue_gathers(b)

    plsc.subcore_barrier()
    pltpu.sync_copy(acc.at[pl.ds(s * _N_SUB, _N_SUB)],
                    z_hbm.at[pl.ds(c * _N_PAD + s * _N_SUB, _N_SUB)])


@functools.cache
def _build_sc_agg():
    mesh = plsc.VectorSubcoreMesh(
        core_axis_name="c", subcore_axis_name="s",
        num_cores=_NCORE, num_subcores=_NSUB)
    return pl.kernel(
        _sc_agg_body,
        out_type=jax.ShapeDtypeStruct((2 * _N_PAD, _H), jnp.float32),
        mesh=mesh,
        compiler_params=pltpu.CompilerParams(use_tc_tiling_on_sc=False),
        scratch_types=[
            pltpu.VMEM((2, _CHUNK_ROWS, _IDX_W), jnp.int32),
            pltpu.VMEM((2, _CHUNK_ROWS, _IDX_W), jnp.int32),
            pltpu.VMEM((2, _CHUNK_E, _H), jnp.float32),
            pltpu.VMEM_SHARED((_ACC_ROWS, _H), jnp.float32),
            pltpu.SemaphoreType.DMA,
            pltpu.SemaphoreType.DMA,
            pltpu.SemaphoreType.DMA,
            pltpu.SemaphoreType.DMA,
        ],
    )


_B_ENC = 10000


def _enc_body(x_ref, pe_ref, w0, b0, w1, b1, w0a, w0b, u_ref):
    h = jnp.maximum(
        jnp.dot(x_ref[...], w0[...], preferred_element_type=jnp.float32)
        + b0[...], 0.0)
    h = jnp.dot(h, w1[...], preferred_element_type=jnp.float32) + b1[...]
    u = (jnp.dot(h, w0a[...], preferred_element_type=jnp.float32)
         + jnp.dot(pe_ref[...], w0b[...], preferred_element_type=jnp.float32))
    u_ref[0, :, :] = u[:, :_H]
    u_ref[1, :, :] = u[:, _H:]


def _full(shape):
    return pl.BlockSpec(shape, lambda i: tuple(0 for _ in shape))


def _tc_encoder(x, pe, w0, b0, w1, b1, w0a, w0b):
    grid = (_N // _B_ENC,)
    return pl.pallas_call(
        _enc_body,
        grid=grid,
        in_specs=[
            pl.BlockSpec((_B_ENC, 37), lambda i: (i, 0)),
            pl.BlockSpec((_B_ENC, 20), lambda i: (i, 0)),
            _full((37, _D)), _full((1, _D)), _full((_D, _D)), _full((1, _D)),
            _full((_D, _D)), _full((20, _D)),
        ],
        out_specs=pl.BlockSpec((2, _B_ENC, _H), lambda i: (0, i, 0)),
        out_shape=jax.ShapeDtypeStruct((2, _N_PAD, _H), jnp.float32),
    )(x, pe, w0, b0, w1, b1, w0a, w0b)


def _mid_body(z_ref, b0, w1, b1, w0n, u_ref):
    z32 = jnp.concatenate([z_ref[0], z_ref[1]], axis=1)
    t = jnp.maximum(z32 + b0[...], 0.0)
    h = jnp.maximum(
        jnp.dot(t, w1[...], preferred_element_type=jnp.float32) + b1[...], 0.0)
    u = jnp.dot(h, w0n[...], preferred_element_type=jnp.float32)
    u_ref[0, :, :] = u[:, :_H]
    u_ref[1, :, :] = u[:, _H:]


def _tc_mid(z, b0, w1, b1, w0n):
    grid = (_N // _B_ENC,)
    return pl.pallas_call(
        _mid_body,
        grid=grid,
        in_specs=[
            pl.BlockSpec((2, _B_ENC, _H), lambda i: (0, i, 0)),
            _full((1, _D)), _full((_D, _D)), _full((1, _D)), _full((_D, _D)),
        ],
        out_specs=pl.BlockSpec((2, _B_ENC, _H), lambda i: (0, i, 0)),
        out_shape=jax.ShapeDtypeStruct((2, _N_PAD, _H), jnp.float32),
    )(z, b0, w1, b1, w0n)


_B_FIN = 2000
_NB_FIN = _N // _B_FIN


def _fin_body(z_ref, batch_ref, b0, w1, b1, ow0, ob0, ow1, ob1, out_ref, acc):
    i = pl.program_id(0)

    @pl.when(i == 0)
    def _():
        acc[...] = jnp.zeros_like(acc)

    z32 = jnp.concatenate([z_ref[0], z_ref[1]], axis=1)
    t = jnp.maximum(z32 + b0[...], 0.0)
    h = jnp.maximum(
        jnp.dot(t, w1[...], preferred_element_type=jnp.float32) + b1[...], 0.0)
    v = jnp.dot(h, ow0[...], preferred_element_type=jnp.float32)
    bvec = batch_ref[0, 0, :]
    onehot = (lax.broadcasted_iota(jnp.int32, (_G, _B_FIN), 0)
              == bvec[None, :]).astype(jnp.float32)
    acc[...] += jnp.dot(onehot, v, preferred_element_type=jnp.float32)

    @pl.when(i == _NB_FIN - 1)
    def _():
        p = jnp.maximum(acc[...] + ob0[...], 0.0)
        out_ref[...] = (jnp.dot(p, ow1[...], preferred_element_type=jnp.float32)
                        + ob1[...])


def _tc_final(z, batch3, b0, w1, b1, ow0, ob0, ow1, ob1):
    return pl.pallas_call(
        _fin_body,
        grid=(_NB_FIN,),
        in_specs=[
            pl.BlockSpec((2, _B_FIN, _H), lambda i: (0, i, 0)),
            pl.BlockSpec((1, 1, _B_FIN), lambda i: (i, 0, 0)),
            _full((1, _D)), _full((_D, _D)), _full((1, _D)), _full((_D, _D)),
            _full((1, _D)), _full((_D, 2)), _full((1, 2)),
        ],
        out_specs=pl.BlockSpec((_G, 2), lambda i: (0, 0)),
        out_shape=jax.ShapeDtypeStruct((_G, 2), jnp.float32),
        scratch_shapes=[pltpu.VMEM((_G, _D), jnp.float32)],
    )(z, batch3, b0, w1, b1, ow0, ob0, ow1, ob1)


def kernel(x, node_pe, edge_index, batch,
           enc_W0, enc_b0, enc_W1, enc_b1,
           gin0_W0, gin0_b0, gin0_W1, gin0_b1,
           gin1_W0, gin1_b0, gin1_W1, gin1_b1,
           gin2_W0, gin2_b0, gin2_W1, gin2_b1,
           out_W0, out_b0, out_W1, out_b1):
    src = edge_index[0]
    dst = edge_index[1]
    pad = _E_PAD - _E
    src_p = jnp.concatenate([src, jnp.zeros((pad,), jnp.int32)])
    dst_p = jnp.concatenate([dst, jnp.full((pad,), _N, jnp.int32)])
    src_idx = jnp.concatenate([src_p, src_p + _N_PAD]).reshape(2 * _ROWS_TOTAL,
                                                           _IDX_W)
    dst_idx = dst_p.reshape(_ROWS_TOTAL, _IDX_W)
    batch3 = batch.reshape(_NB_FIN, 1, _B_FIN)

    r = lambda b: b.reshape(1, -1)
    _sc_agg = _build_sc_agg()

    u = _tc_encoder(x, node_pe, enc_W0, r(enc_b0), enc_W1, r(enc_b1),
                    gin0_W0[:_D], gin0_W0[_D:])
    z = _sc_agg(u.reshape(2 * _N_PAD, _H), src_idx, dst_idx)
    u = _tc_mid(z.reshape(2, _N_PAD, _H), r(gin0_b0), gin0_W1, r(gin0_b1), gin1_W0)
    z = _sc_agg(u.reshape(2 * _N_PAD, _H), src_idx, dst_idx)
    u = _tc_mid(z.reshape(2, _N_PAD, _H), r(gin1_b0), gin1_W1, r(gin1_b1), gin2_W0)
    z = _sc_agg(u.reshape(2 * _N_PAD, _H), src_idx, dst_idx)
    return _tc_final(z.reshape(2, _N_PAD, _H), batch3, r(gin2_b0), gin2_W1,
                     r(gin2_b1), out_W0, r(out_b0), out_W1, r(out_b1))


# speedup vs baseline: 15.5905x; 1.2659x over previous
import functools

import jax
import jax.numpy as jnp
from jax import lax
from jax.experimental import pallas as pl
from jax.experimental.pallas import tpu as pltpu
from jax.experimental.pallas import tpu_sc as plsc

_N = 100000
_E = 1600000
_G = 512
_D = 32
_H = 16
_NSUB = 16
_NCORE = 2

_IDX_W = 128
_CHUNK_ROWS = 4
_CHUNK_E = _CHUNK_ROWS * _IDX_W
_CHUNKS_PER_SUB = 196
_E_SUB = _CHUNK_E * _CHUNKS_PER_SUB
_E_PAD = _E_SUB * _NSUB
_ROWS_PER_SUB = _E_SUB // _IDX_W
_ROWS_TOTAL = _E_PAD // _IDX_W
_N_PAD = 100096
_N_SUB = _N_PAD // _NSUB
_ACC_ROWS = _N_PAD


def _sc_core_work(u_hbm, src_hbm, dst_hbm, z_hbm, srcv, dstv, rowsv, acc,
                  gsems, ssems, s):
    pltpu.sync_copy(u_hbm.at[pl.ds(s * _N_SUB, _N_SUB)],
                    acc.at[pl.ds(s * _N_SUB, _N_SUB)])
    plsc.subcore_barrier()

    def load_idx(k, b):
        base = s * _ROWS_PER_SUB + k * _CHUNK_ROWS
        pltpu.sync_copy(src_hbm.at[pl.ds(base, _CHUNK_ROWS)], srcv.at[b])
        pltpu.sync_copy(dst_hbm.at[pl.ds(base, _CHUNK_ROWS)], dstv.at[b])

    def issue_gathers(b):
        for j in range(_CHUNK_ROWS):
            pltpu.async_copy(u_hbm.at[srcv.at[b].at[j]],
                             rowsv.at[b].at[pl.ds(j * _IDX_W, _IDX_W)],
                             gsems[b])

    def issue_scatters(b):
        for j in range(_CHUNK_ROWS):
            pltpu.async_copy(rowsv.at[b].at[pl.ds(j * _IDX_W, _IDX_W)],
                             acc.at[dstv.at[b].at[j]], ssems[b], add=True)

    def drain(b, sem):
        pltpu.make_async_copy(u_hbm.at[pl.ds(0, _CHUNK_E)], rowsv.at[b],
                              sem).wait()

    for b in (0, 1):
        load_idx(b, b)
        issue_gathers(b)

    @pl.loop(0, _CHUNKS_PER_SUB // 2)
    def _step(t):
        for b in (0, 1):
            drain(b, gsems[b])
            issue_scatters(b)
        for b in (0, 1):
            @pl.when(t < _CHUNKS_PER_SUB // 2 - 1)
            def _():
                load_idx(2 * t + 2 + b, b)
            drain(b, ssems[b])

            @pl.when(t < _CHUNKS_PER_SUB // 2 - 1)
            def _():
                issue_gathers(b)

    plsc.subcore_barrier()
    pltpu.sync_copy(acc.at[pl.ds(s * _N_SUB, _N_SUB)],
                    z_hbm.at[pl.ds(s * _N_SUB, _N_SUB)])


def _sc_agg_body(u_lo, u_hi, src_hbm, dst_hbm, z_lo, z_hi,
                 srcv, dstv, rowsv, acc, gsem0, gsem1, ssem0, ssem1):
    c = lax.axis_index("c")
    s = lax.axis_index("s")
    gsems = (gsem0, gsem1)
    ssems = (ssem0, ssem1)

    @pl.when(c == 0)
    def _():
        _sc_core_work(u_lo, src_hbm, dst_hbm, z_lo, srcv, dstv, rowsv, acc,
                      gsems, ssems, s)

    @pl.when(c == 1)
    def _():
        _sc_core_work(u_hi, src_hbm, dst_hbm, z_hi, srcv, dstv, rowsv, acc,
                      gsems, ssems, s)


@functools.cache
def _build_sc_agg():
    mesh = plsc.VectorSubcoreMesh(
        core_axis_name="c", subcore_axis_name="s",
        num_cores=_NCORE, num_subcores=_NSUB)
    half = jax.ShapeDtypeStruct((_N_PAD, _H), jnp.float32)
    return pl.kernel(
        _sc_agg_body,
        out_type=(half, half),
        mesh=mesh,
        compiler_params=pltpu.CompilerParams(use_tc_tiling_on_sc=False),
        scratch_types=[
            pltpu.VMEM((2, _CHUNK_ROWS, _IDX_W), jnp.int32),
            pltpu.VMEM((2, _CHUNK_ROWS, _IDX_W), jnp.int32),
            pltpu.VMEM((2, _CHUNK_E, _H), jnp.float32),
            pltpu.VMEM_SHARED((_ACC_ROWS, _H), jnp.float32),
            pltpu.SemaphoreType.DMA,
            pltpu.SemaphoreType.DMA,
            pltpu.SemaphoreType.DMA,
            pltpu.SemaphoreType.DMA,
        ],
    )


import numpy as np

_R = _N_PAD // 8
_RB = 736
_GRID = _R // _RB

_a256 = np.arange(256)
_n256 = (_a256 % 128) // 16
_f256 = 16 * (_a256 // 128) + _a256 % 16


def _wbig(W):
    mask = jnp.asarray((_n256[:, None] == _n256[None, :]).astype(np.float32))
    return W[_f256][:, _f256] * mask


def _wbig_in(W, d):
    a = np.arange(8 * d)
    n_in, f_in = a // d, a % d
    mask = jnp.asarray((n_in[:, None] == _n256[None, :]).astype(np.float32))
    return W[f_in][:, _f256] * mask


def _bbig(b):
    return b[_f256].reshape(1, 256)


def _full(shape):
    return pl.BlockSpec(shape, lambda i: tuple(0 for _ in shape))


def _packed_struct():
    return jax.ShapeDtypeStruct((_R, 128), jnp.float32)


def _enc_body(xp_ref, pep_ref, w0b, b0b, w1b, b1b, w0ab, w0bb, ulo_ref,
              uhi_ref):
    h = jnp.maximum(
        jnp.dot(xp_ref[...], w0b[...], preferred_element_type=jnp.float32)
        + b0b[...], 0.0)
    h = jnp.dot(h, w1b[...], preferred_element_type=jnp.float32) + b1b[...]
    u = (jnp.dot(h, w0ab[...], preferred_element_type=jnp.float32)
         + jnp.dot(pep_ref[...], w0bb[...], preferred_element_type=jnp.float32))
    ulo_ref[...] = u[:, :128]
    uhi_ref[...] = u[:, 128:]


def _tc_encoder(xp, pep, w0b, b0b, w1b, b1b, w0ab, w0bb):
    return pl.pallas_call(
        _enc_body,
        grid=(_GRID,),
        in_specs=[
            pl.BlockSpec((_RB, 8 * 37), lambda i: (i, 0)),
            pl.BlockSpec((_RB, 8 * 20), lambda i: (i, 0)),
            _full((8 * 37, 256)), _full((1, 256)), _full((256, 256)),
            _full((1, 256)), _full((256, 256)), _full((8 * 20, 256)),
        ],
        out_specs=[pl.BlockSpec((_RB, 128), lambda i: (i, 0)),
                   pl.BlockSpec((_RB, 128), lambda i: (i, 0))],
        out_shape=[_packed_struct(), _packed_struct()],
    )(xp, pep, w0b, b0b, w1b, b1b, w0ab, w0bb)


def _mid_body(zlo_ref, zhi_ref, b0b, w1b, b1b, w0nb, ulo_ref, uhi_ref):
    zp = jnp.concatenate([zlo_ref[...], zhi_ref[...]], axis=1)
    t = jnp.maximum(zp + b0b[...], 0.0)
    h = jnp.maximum(
        jnp.dot(t, w1b[...], preferred_element_type=jnp.float32) + b1b[...],
        0.0)
    u = jnp.dot(h, w0nb[...], preferred_element_type=jnp.float32)
    ulo_ref[...] = u[:, :128]
    uhi_ref[...] = u[:, 128:]


def _tc_mid(z_lo, z_hi, b0b, w1b, b1b, w0nb):
    return pl.pallas_call(
        _mid_body,
        grid=(_GRID,),
        in_specs=[
            pl.BlockSpec((_RB, 128), lambda i: (i, 0)),
            pl.BlockSpec((_RB, 128), lambda i: (i, 0)),
            _full((1, 256)), _full((256, 256)), _full((1, 256)),
            _full((256, 256)),
        ],
        out_specs=[pl.BlockSpec((_RB, 128), lambda i: (i, 0)),
                   pl.BlockSpec((_RB, 128), lambda i: (i, 0))],
        out_shape=[_packed_struct(), _packed_struct()],
    )(z_lo, z_hi, b0b, w1b, b1b, w0nb)


def _fin_body(zlo_ref, zhi_ref, b8_ref, b0b, w1b, b1b, ow0b, ob0, ow1, ob1,
              out_ref, acc):
    i = pl.program_id(0)

    @pl.when(i == 0)
    def _():
        acc[...] = jnp.zeros_like(acc)

    zp = jnp.concatenate([zlo_ref[...], zhi_ref[...]], axis=1)
    t = jnp.maximum(zp + b0b[...], 0.0)
    h = jnp.maximum(
        jnp.dot(t, w1b[...], preferred_element_type=jnp.float32) + b1b[...],
        0.0)
    vp = jnp.dot(h, ow0b[...], preferred_element_type=jnp.float32)
    for n in range(8):
        bv = b8_ref[0, n, :]
        onehot = (lax.broadcasted_iota(jnp.int32, (_G, _RB), 0)
                  == bv[None, :]).astype(jnp.float32)
        vn = jnp.concatenate([vp[:, n * 16:(n + 1) * 16],
                              vp[:, 128 + n * 16:128 + (n + 1) * 16]], axis=1)
        acc[...] += jnp.dot(onehot, vn, preferred_element_type=jnp.float32)

    @pl.when(i == _GRID - 1)
    def _():
        p = jnp.maximum(acc[...] + ob0[...], 0.0)
        out_ref[...] = (jnp.dot(p, ow1[...], preferred_element_type=jnp.float32)
                        + ob1[...])


def _tc_final(z_lo, z_hi, b8, b0b, w1b, b1b, ow0b, ob0, ow1, ob1):
    return pl.pallas_call(
        _fin_body,
        grid=(_GRID,),
        in_specs=[
            pl.BlockSpec((_RB, 128), lambda i: (i, 0)),
            pl.BlockSpec((_RB, 128), lambda i: (i, 0)),
            pl.BlockSpec((1, 8, _RB), lambda i: (i, 0, 0)),
            _full((1, 256)), _full((256, 256)), _full((1, 256)),
            _full((256, 256)), _full((1, _D)), _full((_D, 2)), _full((1, 2)),
        ],
        out_specs=pl.BlockSpec((_G, 2), lambda i: (0, 0)),
        out_shape=jax.ShapeDtypeStruct((_G, 2), jnp.float32),
        scratch_shapes=[pltpu.VMEM((_G, _D), jnp.float32)],
    )(z_lo, z_hi, b8, b0b, w1b, b1b, ow0b, ob0, ow1, ob1)


def kernel(x, node_pe, edge_index, batch,
           enc_W0, enc_b0, enc_W1, enc_b1,
           gin0_W0, gin0_b0, gin0_W1, gin0_b1,
           gin1_W0, gin1_b0, gin1_W1, gin1_b1,
           gin2_W0, gin2_b0, gin2_W1, gin2_b1,
           out_W0, out_b0, out_W1, out_b1):
    src = edge_index[0]
    dst = edge_index[1]
    pad = _E_PAD - _E
    src_p = jnp.concatenate([src, jnp.zeros((pad,), jnp.int32)])
    dst_p = jnp.concatenate([dst, jnp.full((pad,), _N, jnp.int32)])
    src_idx = src_p.reshape(_ROWS_TOTAL, _IDX_W)
    dst_idx = dst_p.reshape(_ROWS_TOTAL, _IDX_W)

    xp = jnp.pad(x, ((0, _N_PAD - _N), (0, 0))).reshape(_R, 8 * 37)
    pep = jnp.pad(node_pe, ((0, _N_PAD - _N), (0, 0))).reshape(_R, 8 * 20)
    b8 = jnp.pad(batch, (0, _N_PAD - _N),
                 constant_values=_G).reshape(_GRID, _RB, 8).transpose(0, 2, 1)

    _sc_agg = _build_sc_agg()
    packed = lambda a: a.reshape(_R, 128)
    flat = lambda a: a.reshape(_N_PAD, _H)

    u_lo, u_hi = _tc_encoder(
        xp, pep, _wbig_in(enc_W0, 37), _bbig(enc_b0), _wbig(enc_W1),
        _bbig(enc_b1), _wbig(gin0_W0[:_D]), _wbig_in(gin0_W0[_D:], 20))
    z_lo, z_hi = _sc_agg(flat(u_lo), flat(u_hi), src_idx, dst_idx)
    u_lo, u_hi = _tc_mid(packed(z_lo), packed(z_hi), _bbig(gin0_b0),
                         _wbig(gin0_W1), _bbig(gin0_b1), _wbig(gin1_W0))
    z_lo, z_hi = _sc_agg(flat(u_lo), flat(u_hi), src_idx, dst_idx)
    u_lo, u_hi = _tc_mid(packed(z_lo), packed(z_hi), _bbig(gin1_b0),
                         _wbig(gin1_W1), _bbig(gin1_b1), _wbig(gin2_W0))
    z_lo, z_hi = _sc_agg(flat(u_lo), flat(u_hi), src_idx, dst_idx)
    return _tc_final(packed(z_lo), packed(z_hi), b8, _bbig(gin2_b0),
                     _wbig(gin2_W1), _bbig(gin2_b1), _wbig(out_W0),
                     out_b0.reshape(1, -1), out_W1, out_b1.reshape(1, -1))


# speedup vs baseline: 18.2473x; 1.1704x over previous
import functools

import jax
import jax.numpy as jnp
from jax import lax
from jax.experimental import pallas as pl
from jax.experimental.pallas import tpu as pltpu
from jax.experimental.pallas import tpu_sc as plsc

_N = 100000
_E = 1600000
_G = 512
_D = 32
_H = 16
_NSUB = 16
_NCORE = 2

_IDX_W = 128
_CHUNK_ROWS = 4
_CHUNK_E = _CHUNK_ROWS * _IDX_W
_NBUF = 3
_CHUNKS_PER_SUB = 198
_E_SUB = _CHUNK_E * _CHUNKS_PER_SUB
_E_PAD = _E_SUB * _NSUB
_ROWS_PER_SUB = _E_SUB // _IDX_W
_ROWS_TOTAL = _E_PAD // _IDX_W
_N_PAD = 100096
_N_SUB = _N_PAD // _NSUB
_ACC_ROWS = _N_PAD


def _sc_core_work(u_hbm, src_hbm, dst_hbm, z_hbm, srcv, dstv, rowsv, acc,
                  gsems, ssems, isems, s):
    pltpu.sync_copy(u_hbm.at[pl.ds(s * _N_SUB, _N_SUB)],
                    acc.at[pl.ds(s * _N_SUB, _N_SUB)])
    plsc.subcore_barrier()

    def issue_idx(k, b6):
        base = s * _ROWS_PER_SUB + k * _CHUNK_ROWS
        pltpu.async_copy(src_hbm.at[pl.ds(base, _CHUNK_ROWS)], srcv.at[b6],
                         isems[b6])
        pltpu.async_copy(dst_hbm.at[pl.ds(base, _CHUNK_ROWS)], dstv.at[b6],
                         isems[b6])

    def drain_idx(b6):
        pltpu.make_async_copy(src_hbm.at[pl.ds(0, _CHUNK_ROWS)], srcv.at[b6],
                              isems[b6]).wait()
        pltpu.make_async_copy(dst_hbm.at[pl.ds(0, _CHUNK_ROWS)], dstv.at[b6],
                              isems[b6]).wait()

    def issue_gathers(b, b6):
        for j in range(_CHUNK_ROWS):
            pltpu.async_copy(u_hbm.at[srcv.at[b6].at[j]],
                             rowsv.at[b].at[pl.ds(j * _IDX_W, _IDX_W)],
                             gsems[b])

    def issue_scatters(b, b6):
        for j in range(_CHUNK_ROWS):
            pltpu.async_copy(rowsv.at[b].at[pl.ds(j * _IDX_W, _IDX_W)],
                             acc.at[dstv.at[b6].at[j]], ssems[b], add=True)

    def drain_rows(b, sem):
        pltpu.make_async_copy(u_hbm.at[pl.ds(0, _CHUNK_E)], rowsv.at[b],
                              sem).wait()

    for k in range(4):
        issue_idx(k, k)
    for k in range(2):
        drain_idx(k)
        issue_gathers(k, k)

    @pl.loop(0, _CHUNKS_PER_SUB // 6)
    def _step(t):
        for i in range(6):
            k = 6 * t + i
            b = i % 3
            b6 = i
            bn = (i + 2) % 3
            drain_rows(b, gsems[b])
            issue_scatters(b, b6)

            @pl.when(k >= 1)
            def _():
                drain_rows(bn, ssems[bn])

            @pl.when(k + 2 < _CHUNKS_PER_SUB)
            def _():
                drain_idx((i + 2) % 6)
                issue_gathers(bn, (i + 2) % 6)

            @pl.when(k + 4 < _CHUNKS_PER_SUB)
            def _():
                issue_idx(k + 4, (i + 4) % 6)

    drain_rows((_CHUNKS_PER_SUB - 1) % 3, ssems[(_CHUNKS_PER_SUB - 1) % 3])
    plsc.subcore_barrier()
    pltpu.sync_copy(acc.at[pl.ds(s * _N_SUB, _N_SUB)],
                    z_hbm.at[pl.ds(s * _N_SUB, _N_SUB)])


def _sc_agg_body(u_lo, u_hi, src_hbm, dst_hbm, z_lo, z_hi,
                 srcv, dstv, rowsv, acc, *sems):
    c = lax.axis_index("c")
    s = lax.axis_index("s")
    gsems = sems[0:3]
    ssems = sems[3:6]
    isems = sems[6:12]

    @pl.when(c == 0)
    def _():
        _sc_core_work(u_lo, src_hbm, dst_hbm, z_lo, srcv, dstv, rowsv, acc,
                      gsems, ssems, isems, s)

    @pl.when(c == 1)
    def _():
        _sc_core_work(u_hi, src_hbm, dst_hbm, z_hi, srcv, dstv, rowsv, acc,
                      gsems, ssems, isems, s)


@functools.cache
def _build_sc_agg():
    mesh = plsc.VectorSubcoreMesh(
        core_axis_name="c", subcore_axis_name="s",
        num_cores=_NCORE, num_subcores=_NSUB)
    half = jax.ShapeDtypeStruct((_N_PAD, _H), jnp.float32)
    return pl.kernel(
        _sc_agg_body,
        out_type=(half, half),
        mesh=mesh,
        compiler_params=pltpu.CompilerParams(use_tc_tiling_on_sc=False),
        scratch_types=[
            pltpu.VMEM((6, _CHUNK_ROWS, _IDX_W), jnp.int32),
            pltpu.VMEM((6, _CHUNK_ROWS, _IDX_W), jnp.int32),
            pltpu.VMEM((_NBUF, _CHUNK_E, _H), jnp.float32),
            pltpu.VMEM_SHARED((_ACC_ROWS, _H), jnp.float32),
        ] + [pltpu.SemaphoreType.DMA] * 12,
    )


import numpy as np

_R = _N_PAD // 8
_RB = 736
_GRID = _R // _RB

_a256 = np.arange(256)
_n256 = (_a256 % 128) // 16
_f256 = 16 * (_a256 // 128) + _a256 % 16


def _wbig(W):
    mask = jnp.asarray((_n256[:, None] == _n256[None, :]).astype(np.float32))
    return W[_f256][:, _f256] * mask


def _wbig_in(W, d):
    a = np.arange(8 * d)
    n_in, f_in = a // d, a % d
    mask = jnp.asarray((n_in[:, None] == _n256[None, :]).astype(np.float32))
    return W[f_in][:, _f256] * mask


def _bbig(b):
    return b[_f256].reshape(1, 256)


def _full(shape):
    return pl.BlockSpec(shape, lambda i: tuple(0 for _ in shape))


def _packed_struct():
    return jax.ShapeDtypeStruct((_R, 128), jnp.float32)


def _enc_body(xp_ref, pep_ref, w0b, b0b, w1b, b1b, w0ab, w0bb, ulo_ref,
              uhi_ref):
    h = jnp.maximum(
        jnp.dot(xp_ref[...], w0b[...], preferred_element_type=jnp.float32)
        + b0b[...], 0.0)
    h = jnp.dot(h, w1b[...], preferred_element_type=jnp.float32) + b1b[...]
    u = (jnp.dot(h, w0ab[...], preferred_element_type=jnp.float32)
         + jnp.dot(pep_ref[...], w0bb[...], preferred_element_type=jnp.float32))
    ulo_ref[...] = u[:, :128]
    uhi_ref[...] = u[:, 128:]


def _tc_encoder(xp, pep, w0b, b0b, w1b, b1b, w0ab, w0bb):
    return pl.pallas_call(
        _enc_body,
        grid=(_GRID,),
        in_specs=[
            pl.BlockSpec((_RB, 8 * 37), lambda i: (i, 0)),
            pl.BlockSpec((_RB, 8 * 20), lambda i: (i, 0)),
            _full((8 * 37, 256)), _full((1, 256)), _full((256, 256)),
            _full((1, 256)), _full((256, 256)), _full((8 * 20, 256)),
        ],
        out_specs=[pl.BlockSpec((_RB, 128), lambda i: (i, 0)),
                   pl.BlockSpec((_RB, 128), lambda i: (i, 0))],
        out_shape=[_packed_struct(), _packed_struct()],
    )(xp, pep, w0b, b0b, w1b, b1b, w0ab, w0bb)


def _mid_body(zlo_ref, zhi_ref, b0b, w1b, b1b, w0nb, ulo_ref, uhi_ref):
    zp = jnp.concatenate([zlo_ref[...], zhi_ref[...]], axis=1)
    t = jnp.maximum(zp + b0b[...], 0.0)
    h = jnp.maximum(
        jnp.dot(t, w1b[...], preferred_element_type=jnp.float32) + b1b[...],
        0.0)
    u = jnp.dot(h, w0nb[...], preferred_element_type=jnp.float32)
    ulo_ref[...] = u[:, :128]
    uhi_ref[...] = u[:, 128:]


def _tc_mid(z_lo, z_hi, b0b, w1b, b1b, w0nb):
    return pl.pallas_call(
        _mid_body,
        grid=(_GRID,),
        in_specs=[
            pl.BlockSpec((_RB, 128), lambda i: (i, 0)),
            pl.BlockSpec((_RB, 128), lambda i: (i, 0)),
            _full((1, 256)), _full((256, 256)), _full((1, 256)),
            _full((256, 256)),
        ],
        out_specs=[pl.BlockSpec((_RB, 128), lambda i: (i, 0)),
                   pl.BlockSpec((_RB, 128), lambda i: (i, 0))],
        out_shape=[_packed_struct(), _packed_struct()],
    )(z_lo, z_hi, b0b, w1b, b1b, w0nb)


def _fin_body(zlo_ref, zhi_ref, b8_ref, b0b, w1b, b1b, ow0b, ob0, ow1, ob1,
              out_ref, acc):
    i = pl.program_id(0)

    @pl.when(i == 0)
    def _():
        acc[...] = jnp.zeros_like(acc)

    zp = jnp.concatenate([zlo_ref[...], zhi_ref[...]], axis=1)
    t = jnp.maximum(zp + b0b[...], 0.0)
    h = jnp.maximum(
        jnp.dot(t, w1b[...], preferred_element_type=jnp.float32) + b1b[...],
        0.0)
    vp = jnp.dot(h, ow0b[...], preferred_element_type=jnp.float32)
    for n in range(8):
        bv = b8_ref[0, n, :]
        onehot = (lax.broadcasted_iota(jnp.int32, (_G, _RB), 0)
                  == bv[None, :]).astype(jnp.float32)
        vn = jnp.concatenate([vp[:, n * 16:(n + 1) * 16],
                              vp[:, 128 + n * 16:128 + (n + 1) * 16]], axis=1)
        acc[...] += jnp.dot(onehot, vn, preferred_element_type=jnp.float32)

    @pl.when(i == _GRID - 1)
    def _():
        p = jnp.maximum(acc[...] + ob0[...], 0.0)
        out_ref[...] = (jnp.dot(p, ow1[...], preferred_element_type=jnp.float32)
                        + ob1[...])


def _tc_final(z_lo, z_hi, b8, b0b, w1b, b1b, ow0b, ob0, ow1, ob1):
    return pl.pallas_call(
        _fin_body,
        grid=(_GRID,),
        in_specs=[
            pl.BlockSpec((_RB, 128), lambda i: (i, 0)),
            pl.BlockSpec((_RB, 128), lambda i: (i, 0)),
            pl.BlockSpec((1, 8, _RB), lambda i: (i, 0, 0)),
            _full((1, 256)), _full((256, 256)), _full((1, 256)),
            _full((256, 256)), _full((1, _D)), _full((_D, 2)), _full((1, 2)),
        ],
        out_specs=pl.BlockSpec((_G, 2), lambda i: (0, 0)),
        out_shape=jax.ShapeDtypeStruct((_G, 2), jnp.float32),
        scratch_shapes=[pltpu.VMEM((_G, _D), jnp.float32)],
    )(z_lo, z_hi, b8, b0b, w1b, b1b, ow0b, ob0, ow1, ob1)


def kernel(x, node_pe, edge_index, batch,
           enc_W0, enc_b0, enc_W1, enc_b1,
           gin0_W0, gin0_b0, gin0_W1, gin0_b1,
           gin1_W0, gin1_b0, gin1_W1, gin1_b1,
           gin2_W0, gin2_b0, gin2_W1, gin2_b1,
           out_W0, out_b0, out_W1, out_b1):
    src = edge_index[0]
    dst = edge_index[1]
    pad = _E_PAD - _E
    src_p = jnp.concatenate([src, jnp.zeros((pad,), jnp.int32)])
    dst_p = jnp.concatenate([dst, jnp.full((pad,), _N, jnp.int32)])
    src_idx = src_p.reshape(_ROWS_TOTAL, _IDX_W)
    dst_idx = dst_p.reshape(_ROWS_TOTAL, _IDX_W)

    xp = jnp.pad(x, ((0, _N_PAD - _N), (0, 0))).reshape(_R, 8 * 37)
    pep = jnp.pad(node_pe, ((0, _N_PAD - _N), (0, 0))).reshape(_R, 8 * 20)
    b8 = jnp.pad(batch, (0, _N_PAD - _N),
                 constant_values=_G).reshape(_GRID, _RB, 8).transpose(0, 2, 1)

    _sc_agg = _build_sc_agg()
    packed = lambda a: a.reshape(_R, 128)
    flat = lambda a: a.reshape(_N_PAD, _H)

    u_lo, u_hi = _tc_encoder(
        xp, pep, _wbig_in(enc_W0, 37), _bbig(enc_b0), _wbig(enc_W1),
        _bbig(enc_b1), _wbig(gin0_W0[:_D]), _wbig_in(gin0_W0[_D:], 20))
    z_lo, z_hi = _sc_agg(flat(u_lo), flat(u_hi), src_idx, dst_idx)
    u_lo, u_hi = _tc_mid(packed(z_lo), packed(z_hi), _bbig(gin0_b0),
                         _wbig(gin0_W1), _bbig(gin0_b1), _wbig(gin1_W0))
    z_lo, z_hi = _sc_agg(flat(u_lo), flat(u_hi), src_idx, dst_idx)
    u_lo, u_hi = _tc_mid(packed(z_lo), packed(z_hi), _bbig(gin1_b0),
                         _wbig(gin1_W1), _bbig(gin1_b1), _wbig(gin2_W0))
    z_lo, z_hi = _sc_agg(flat(u_lo), flat(u_hi), src_idx, dst_idx)
    return _tc_final(packed(z_lo), packed(z_hi), b8, _bbig(gin2_b0),
                     _wbig(gin2_W1), _bbig(gin2_b1), _wbig(out_W0),
                     out_b0.reshape(1, -1), out_W1, out_b1.reshape(1, -1))


# speedup vs baseline: 18.6965x; 1.0246x over previous
import functools

import jax
import jax.numpy as jnp
from jax import lax
from jax.experimental import pallas as pl
from jax.experimental.pallas import tpu as pltpu
from jax.experimental.pallas import tpu_sc as plsc

_N = 100000
_E = 1600000
_G = 512
_D = 32
_H = 16
_NSUB = 16
_NCORE = 2

_IDX_W = 128
_CHUNK_ROWS = 4
_CHUNK_E = _CHUNK_ROWS * _IDX_W
_NBUF = 3
_CHUNKS_PER_SUB = 198
_E_SUB = _CHUNK_E * _CHUNKS_PER_SUB
_E_PAD = _E_SUB * _NSUB
_ROWS_PER_SUB = _E_SUB // _IDX_W
_ROWS_TOTAL = _E_PAD // _IDX_W
_ROWS_MAIN = _E // _IDX_W
_ROWS_PAD = _ROWS_TOTAL - _ROWS_MAIN
_N_PAD = 100096
_N_SUB = _N_PAD // _NSUB
_ACC_ROWS = _N_PAD


def _sc_core_work(u_hbm, src_hbm, dst_hbm, srcp_hbm, dstp_hbm, z_hbm,
                  srcv, dstv, rowsv, acc, gsems, ssems, isems, s):
    pltpu.sync_copy(u_hbm.at[pl.ds(s * _N_SUB, _N_SUB)],
                    acc.at[pl.ds(s * _N_SUB, _N_SUB)])
    plsc.subcore_barrier()

    def issue_idx(k, b6):
        base = s * _ROWS_PER_SUB + k * _CHUNK_ROWS

        @pl.when(base < _ROWS_MAIN)
        def _():
            pltpu.async_copy(src_hbm.at[pl.ds(base, _CHUNK_ROWS)],
                             srcv.at[b6], isems[b6])
            pltpu.async_copy(dst_hbm.at[pl.ds(base, _CHUNK_ROWS)],
                             dstv.at[b6], isems[b6])

        @pl.when(base >= _ROWS_MAIN)
        def _():
            pb = base - _ROWS_MAIN
            pltpu.async_copy(srcp_hbm.at[pl.ds(pb, _CHUNK_ROWS)],
                             srcv.at[b6], isems[b6])
            pltpu.async_copy(dstp_hbm.at[pl.ds(pb, _CHUNK_ROWS)],
                             dstv.at[b6], isems[b6])

    def drain_idx(b6):
        pltpu.make_async_copy(src_hbm.at[pl.ds(0, _CHUNK_ROWS)], srcv.at[b6],
                              isems[b6]).wait()
        pltpu.make_async_copy(dst_hbm.at[pl.ds(0, _CHUNK_ROWS)], dstv.at[b6],
                              isems[b6]).wait()

    def issue_gathers(b, b6):
        for j in range(_CHUNK_ROWS):
            pltpu.async_copy(u_hbm.at[srcv.at[b6].at[j]],
                             rowsv.at[b].at[pl.ds(j * _IDX_W, _IDX_W)],
                             gsems[b])

    def issue_scatters(b, b6):
        for j in range(_CHUNK_ROWS):
            pltpu.async_copy(rowsv.at[b].at[pl.ds(j * _IDX_W, _IDX_W)],
                             acc.at[dstv.at[b6].at[j]], ssems[b], add=True)

    def drain_rows(b, sem):
        pltpu.make_async_copy(u_hbm.at[pl.ds(0, _CHUNK_E)], rowsv.at[b],
                              sem).wait()

    for k in range(4):
        issue_idx(k, k)
    for k in range(2):
        drain_idx(k)
        issue_gathers(k, k)

    @pl.loop(0, _CHUNKS_PER_SUB // 6)
    def _step(t):
        for i in range(6):
            k = 6 * t + i
            b = i % 3
            b6 = i
            bn = (i + 2) % 3
            drain_rows(b, gsems[b])
            issue_scatters(b, b6)

            @pl.when(k >= 1)
            def _():
                drain_rows(bn, ssems[bn])

            @pl.when(k + 2 < _CHUNKS_PER_SUB)
            def _():
                drain_idx((i + 2) % 6)
                issue_gathers(bn, (i + 2) % 6)

            @pl.when(k + 4 < _CHUNKS_PER_SUB)
            def _():
                issue_idx(k + 4, (i + 4) % 6)

    drain_rows((_CHUNKS_PER_SUB - 1) % 3, ssems[(_CHUNKS_PER_SUB - 1) % 3])
    plsc.subcore_barrier()
    pltpu.sync_copy(acc.at[pl.ds(s * _N_SUB, _N_SUB)],
                    z_hbm.at[pl.ds(s * _N_SUB, _N_SUB)])


def _sc_agg_body(u_lo, u_hi, src_hbm, dst_hbm, srcp_hbm, dstp_hbm,
                 z_lo, z_hi, srcv, dstv, rowsv, acc, *sems):
    c = lax.axis_index("c")
    s = lax.axis_index("s")
    gsems = sems[0:3]
    ssems = sems[3:6]
    isems = sems[6:12]

    @pl.when(c == 0)
    def _():
        _sc_core_work(u_lo, src_hbm, dst_hbm, srcp_hbm, dstp_hbm, z_lo,
                      srcv, dstv, rowsv, acc, gsems, ssems, isems, s)

    @pl.when(c == 1)
    def _():
        _sc_core_work(u_hi, src_hbm, dst_hbm, srcp_hbm, dstp_hbm, z_hi,
                      srcv, dstv, rowsv, acc, gsems, ssems, isems, s)


@functools.cache
def _build_sc_agg():
    mesh = plsc.VectorSubcoreMesh(
        core_axis_name="c", subcore_axis_name="s",
        num_cores=_NCORE, num_subcores=_NSUB)
    half = jax.ShapeDtypeStruct((_N_PAD, _H), jnp.float32)
    return pl.kernel(
        _sc_agg_body,
        out_type=(half, half),
        mesh=mesh,
        compiler_params=pltpu.CompilerParams(use_tc_tiling_on_sc=False),
        scratch_types=[
            pltpu.VMEM((6, _CHUNK_ROWS, _IDX_W), jnp.int32),
            pltpu.VMEM((6, _CHUNK_ROWS, _IDX_W), jnp.int32),
            pltpu.VMEM((_NBUF, _CHUNK_E, _H), jnp.float32),
            pltpu.VMEM_SHARED((_ACC_ROWS, _H), jnp.float32),
        ] + [pltpu.SemaphoreType.DMA] * 12,
    )


import numpy as np

_R = _N_PAD // 8
_RB = 736
_GRID = _R // _RB

_a256 = np.arange(256)
_n256 = (_a256 % 128) // 16
_f256 = 16 * (_a256 // 128) + _a256 % 16


def _wbig(W):
    mask = jnp.asarray((_n256[:, None] == _n256[None, :]).astype(np.float32))
    return W[_f256][:, _f256] * mask


def _wbig_in(W, d):
    a = np.arange(8 * d)
    n_in, f_in = a // d, a % d
    mask = jnp.asarray((n_in[:, None] == _n256[None, :]).astype(np.float32))
    return W[f_in][:, _f256] * mask


def _bbig(b):
    return b[_f256].reshape(1, 256)


def _full(shape):
    return pl.BlockSpec(shape, lambda i: tuple(0 for _ in shape))


def _packed_struct():
    return jax.ShapeDtypeStruct((_R, 128), jnp.float32)


def _enc_body(xp_ref, pep_ref, w0b, b0b, w1b, b1b, w0ab, w0bb, ulo_ref,
              uhi_ref):
    h = jnp.maximum(
        jnp.dot(xp_ref[...], w0b[...], preferred_element_type=jnp.float32)
        + b0b[...], 0.0)
    h = jnp.dot(h, w1b[...], preferred_element_type=jnp.float32) + b1b[...]
    u = (jnp.dot(h, w0ab[...], preferred_element_type=jnp.float32)
         + jnp.dot(pep_ref[...], w0bb[...], preferred_element_type=jnp.float32))
    ulo_ref[...] = u[:, :128]
    uhi_ref[...] = u[:, 128:]


def _tc_encoder(xp, pep, w0b, b0b, w1b, b1b, w0ab, w0bb):
    return pl.pallas_call(
        _enc_body,
        grid=(_GRID,),
        in_specs=[
            pl.BlockSpec((_RB, 8 * 37), lambda i: (i, 0)),
            pl.BlockSpec((_RB, 8 * 20), lambda i: (i, 0)),
            _full((8 * 37, 256)), _full((1, 256)), _full((256, 256)),
            _full((1, 256)), _full((256, 256)), _full((8 * 20, 256)),
        ],
        out_specs=[pl.BlockSpec((_RB, 128), lambda i: (i, 0)),
                   pl.BlockSpec((_RB, 128), lambda i: (i, 0))],
        out_shape=[_packed_struct(), _packed_struct()],
    )(xp, pep, w0b, b0b, w1b, b1b, w0ab, w0bb)


def _mid_body(zlo_ref, zhi_ref, b0b, w1b, b1b, w0nb, ulo_ref, uhi_ref):
    zp = jnp.concatenate([zlo_ref[...], zhi_ref[...]], axis=1)
    t = jnp.maximum(zp + b0b[...], 0.0)
    h = jnp.maximum(
        jnp.dot(t, w1b[...], preferred_element_type=jnp.float32) + b1b[...],
        0.0)
    u = jnp.dot(h, w0nb[...], preferred_element_type=jnp.float32)
    ulo_ref[...] = u[:, :128]
    uhi_ref[...] = u[:, 128:]


def _tc_mid(z_lo, z_hi, b0b, w1b, b1b, w0nb):
    return pl.pallas_call(
        _mid_body,
        grid=(_GRID,),
        in_specs=[
            pl.BlockSpec((_RB, 128), lambda i: (i, 0)),
            pl.BlockSpec((_RB, 128), lambda i: (i, 0)),
            _full((1, 256)), _full((256, 256)), _full((1, 256)),
            _full((256, 256)),
        ],
        out_specs=[pl.BlockSpec((_RB, 128), lambda i: (i, 0)),
                   pl.BlockSpec((_RB, 128), lambda i: (i, 0))],
        out_shape=[_packed_struct(), _packed_struct()],
    )(z_lo, z_hi, b0b, w1b, b1b, w0nb)


def _fin_body(zlo_ref, zhi_ref, b8_ref, b0b, w1b, b1b, ow0b, ob0, ow1, ob1,
              out_ref, acc):
    i = pl.program_id(0)

    @pl.when(i == 0)
    def _():
        acc[...] = jnp.zeros_like(acc)

    zp = jnp.concatenate([zlo_ref[...], zhi_ref[...]], axis=1)
    t = jnp.maximum(zp + b0b[...], 0.0)
    h = jnp.maximum(
        jnp.dot(t, w1b[...], preferred_element_type=jnp.float32) + b1b[...],
        0.0)
    vp = jnp.dot(h, ow0b[...], preferred_element_type=jnp.float32)
    for n in range(8):
        bv = b8_ref[0, n, :]
        onehot = (lax.broadcasted_iota(jnp.int32, (_G, _RB), 0)
                  == bv[None, :]).astype(jnp.float32)
        vn = jnp.concatenate([vp[:, n * 16:(n + 1) * 16],
                              vp[:, 128 + n * 16:128 + (n + 1) * 16]], axis=1)
        acc[...] += jnp.dot(onehot, vn, preferred_element_type=jnp.float32)

    @pl.when(i == _GRID - 1)
    def _():
        p = jnp.maximum(acc[...] + ob0[...], 0.0)
        out_ref[...] = (jnp.dot(p, ow1[...], preferred_element_type=jnp.float32)
                        + ob1[...])


def _tc_final(z_lo, z_hi, b8, b0b, w1b, b1b, ow0b, ob0, ow1, ob1):
    return pl.pallas_call(
        _fin_body,
        grid=(_GRID,),
        in_specs=[
            pl.BlockSpec((_RB, 128), lambda i: (i, 0)),
            pl.BlockSpec((_RB, 128), lambda i: (i, 0)),
            pl.BlockSpec((1, 8, _RB), lambda i: (i, 0, 0)),
            _full((1, 256)), _full((256, 256)), _full((1, 256)),
            _full((256, 256)), _full((1, _D)), _full((_D, 2)), _full((1, 2)),
        ],
        out_specs=pl.BlockSpec((_G, 2), lambda i: (0, 0)),
        out_shape=jax.ShapeDtypeStruct((_G, 2), jnp.float32),
        scratch_shapes=[pltpu.VMEM((_G, _D), jnp.float32)],
    )(z_lo, z_hi, b8, b0b, w1b, b1b, ow0b, ob0, ow1, ob1)


def kernel(x, node_pe, edge_index, batch,
           enc_W0, enc_b0, enc_W1, enc_b1,
           gin0_W0, gin0_b0, gin0_W1, gin0_b1,
           gin1_W0, gin1_b0, gin1_W1, gin1_b1,
           gin2_W0, gin2_b0, gin2_W1, gin2_b1,
           out_W0, out_b0, out_W1, out_b1):
    src_idx = edge_index[0].reshape(_ROWS_MAIN, _IDX_W)
    dst_idx = edge_index[1].reshape(_ROWS_MAIN, _IDX_W)
    src_pad = jnp.zeros((_ROWS_PAD, _IDX_W), jnp.int32)
    dst_pad = jnp.full((_ROWS_PAD, _IDX_W), _N, jnp.int32)

    xp = jnp.pad(x, ((0, _N_PAD - _N), (0, 0))).reshape(_R, 8 * 37)
    pep = jnp.pad(node_pe, ((0, _N_PAD - _N), (0, 0))).reshape(_R, 8 * 20)
    b8 = jnp.pad(batch, (0, _N_PAD - _N),
                 constant_values=_G).reshape(_GRID, _RB, 8).transpose(0, 2, 1)

    _sc_agg = _build_sc_agg()
    packed = lambda a: a.reshape(_R, 128)
    flat = lambda a: a.reshape(_N_PAD, _H)

    u_lo, u_hi = _tc_encoder(
        xp, pep, _wbig_in(enc_W0, 37), _bbig(enc_b0), _wbig(enc_W1),
        _bbig(enc_b1), _wbig(gin0_W0[:_D]), _wbig_in(gin0_W0[_D:], 20))
    z_lo, z_hi = _sc_agg(flat(u_lo), flat(u_hi), src_idx, dst_idx,
                         src_pad, dst_pad)
    u_lo, u_hi = _tc_mid(packed(z_lo), packed(z_hi), _bbig(gin0_b0),
                         _wbig(gin0_W1), _bbig(gin0_b1), _wbig(gin1_W0))
    z_lo, z_hi = _sc_agg(flat(u_lo), flat(u_hi), src_idx, dst_idx,
                         src_pad, dst_pad)
    u_lo, u_hi = _tc_mid(packed(z_lo), packed(z_hi), _bbig(gin1_b0),
                         _wbig(gin1_W1), _bbig(gin1_b1), _wbig(gin2_W0))
    z_lo, z_hi = _sc_agg(flat(u_lo), flat(u_hi), src_idx, dst_idx,
                         src_pad, dst_pad)
    return _tc_final(packed(z_lo), packed(z_hi), b8, _bbig(gin2_b0),
                     _wbig(gin2_W1), _bbig(gin2_b1), _wbig(out_W0),
                     out_b0.reshape(1, -1), out_W1, out_b1.reshape(1, -1))


# speedup vs baseline: 18.7004x; 1.0002x over previous
import functools

import jax
import jax.numpy as jnp
from jax import lax
from jax.experimental import pallas as pl
from jax.experimental.pallas import tpu as pltpu
from jax.experimental.pallas import tpu_sc as plsc

_N = 100000
_E = 1600000
_G = 512
_D = 32
_H = 16
_NSUB = 16
_NCORE = 2

_IDX_W = 128
_CHUNK_ROWS = 4
_CHUNK_E = _CHUNK_ROWS * _IDX_W
_NBUF = 3
_CHUNKS_PER_SUB = 198
_E_SUB = _CHUNK_E * _CHUNKS_PER_SUB
_E_PAD = _E_SUB * _NSUB
_ROWS_PER_SUB = _E_SUB // _IDX_W
_ROWS_TOTAL = _E_PAD // _IDX_W
_ROWS_MAIN = _E // _IDX_W
_ROWS_PAD = _ROWS_TOTAL - _ROWS_MAIN
_N_PAD = 100096
_N_SUB = _N_PAD // _NSUB
_ACC_ROWS = _N_PAD


def _sc_core_work(u_hbm, src_hbm, dst_hbm, srcp_hbm, dstp_hbm, z_hbm,
                  srcv, dstv, rowsv, acc, gsems, ssems, isems, s):
    pltpu.sync_copy(u_hbm.at[pl.ds(s * _N_SUB, _N_SUB)],
                    acc.at[pl.ds(s * _N_SUB, _N_SUB)])
    plsc.subcore_barrier()

    def issue_idx(k, b6):
        base = s * _ROWS_PER_SUB + k * _CHUNK_ROWS

        @pl.when(base < _ROWS_MAIN)
        def _():
            pltpu.async_copy(src_hbm.at[pl.ds(base, _CHUNK_ROWS)],
                             srcv.at[b6], isems[b6])
            pltpu.async_copy(dst_hbm.at[pl.ds(base, _CHUNK_ROWS)],
                             dstv.at[b6], isems[b6])

        @pl.when(base >= _ROWS_MAIN)
        def _():
            pb = base - _ROWS_MAIN
            pltpu.async_copy(srcp_hbm.at[pl.ds(pb, _CHUNK_ROWS)],
                             srcv.at[b6], isems[b6])
            pltpu.async_copy(dstp_hbm.at[pl.ds(pb, _CHUNK_ROWS)],
                             dstv.at[b6], isems[b6])

    def drain_idx(b6):
        pltpu.make_async_copy(src_hbm.at[pl.ds(0, _CHUNK_ROWS)], srcv.at[b6],
                              isems[b6]).wait()
        pltpu.make_async_copy(dst_hbm.at[pl.ds(0, _CHUNK_ROWS)], dstv.at[b6],
                              isems[b6]).wait()

    def issue_gathers(b, b6):
        for j in range(_CHUNK_ROWS):
            pltpu.async_copy(u_hbm.at[srcv.at[b6].at[j]],
                             rowsv.at[b].at[pl.ds(j * _IDX_W, _IDX_W)],
                             gsems[b])

    def issue_scatters(b, b6):
        for j in range(_CHUNK_ROWS):
            pltpu.async_copy(rowsv.at[b].at[pl.ds(j * _IDX_W, _IDX_W)],
                             acc.at[dstv.at[b6].at[j]], ssems[b], add=True)

    def drain_rows(b, sem):
        pltpu.make_async_copy(u_hbm.at[pl.ds(0, _CHUNK_E)], rowsv.at[b],
                              sem).wait()

    for k in range(4):
        issue_idx(k, k)
    for k in range(2):
        drain_idx(k)
        issue_gathers(k, k)

    @pl.loop(0, _CHUNKS_PER_SUB // 6)
    def _step(t):
        for i in range(6):
            k = 6 * t + i
            b = i % 3
            b6 = i
            bn = (i + 2) % 3
            drain_rows(b, gsems[b])

            @pl.when(k >= 1)
            def _():
                drain_rows(bn, ssems[bn])

            @pl.when(k + 2 < _CHUNKS_PER_SUB)
            def _():
                drain_idx((i + 2) % 6)
                issue_gathers(bn, (i + 2) % 6)

            issue_scatters(b, b6)

            @pl.when(k + 4 < _CHUNKS_PER_SUB)
            def _():
                issue_idx(k + 4, (i + 4) % 6)

    drain_rows((_CHUNKS_PER_SUB - 1) % 3, ssems[(_CHUNKS_PER_SUB - 1) % 3])
    plsc.subcore_barrier()
    pltpu.sync_copy(acc.at[pl.ds(s * _N_SUB, _N_SUB)],
                    z_hbm.at[pl.ds(s * _N_SUB, _N_SUB)])


def _sc_agg_body(u_lo, u_hi, src_hbm, dst_hbm, srcp_hbm, dstp_hbm,
                 z_lo, z_hi, srcv, dstv, rowsv, acc, *sems):
    c = lax.axis_index("c")
    s = lax.axis_index("s")
    gsems = sems[0:3]
    ssems = sems[3:6]
    isems = sems[6:12]

    @pl.when(c == 0)
    def _():
        _sc_core_work(u_lo, src_hbm, dst_hbm, srcp_hbm, dstp_hbm, z_lo,
                      srcv, dstv, rowsv, acc, gsems, ssems, isems, s)

    @pl.when(c == 1)
    def _():
        _sc_core_work(u_hi, src_hbm, dst_hbm, srcp_hbm, dstp_hbm, z_hi,
                      srcv, dstv, rowsv, acc, gsems, ssems, isems, s)


@functools.cache
def _build_sc_agg():
    mesh = plsc.VectorSubcoreMesh(
        core_axis_name="c", subcore_axis_name="s",
        num_cores=_NCORE, num_subcores=_NSUB)
    half = jax.ShapeDtypeStruct((_N_PAD, _H), jnp.float32)
    return pl.kernel(
        _sc_agg_body,
        out_type=(half, half),
        mesh=mesh,
        compiler_params=pltpu.CompilerParams(use_tc_tiling_on_sc=False),
        scratch_types=[
            pltpu.VMEM((6, _CHUNK_ROWS, _IDX_W), jnp.int32),
            pltpu.VMEM((6, _CHUNK_ROWS, _IDX_W), jnp.int32),
            pltpu.VMEM((_NBUF, _CHUNK_E, _H), jnp.float32),
            pltpu.VMEM_SHARED((_ACC_ROWS, _H), jnp.float32),
        ] + [pltpu.SemaphoreType.DMA] * 12,
    )


import numpy as np

_R = _N_PAD // 8
_RB = 736
_GRID = _R // _RB

_a256 = np.arange(256)
_n256 = (_a256 % 128) // 16
_f256 = 16 * (_a256 // 128) + _a256 % 16


def _wbig(W):
    mask = jnp.asarray((_n256[:, None] == _n256[None, :]).astype(np.float32))
    return W[_f256][:, _f256] * mask


def _wbig_in(W, d):
    a = np.arange(8 * d)
    n_in, f_in = a // d, a % d
    mask = jnp.asarray((n_in[:, None] == _n256[None, :]).astype(np.float32))
    return W[f_in][:, _f256] * mask


def _bbig(b):
    return b[_f256].reshape(1, 256)


def _full(shape):
    return pl.BlockSpec(shape, lambda i: tuple(0 for _ in shape))


def _packed_struct():
    return jax.ShapeDtypeStruct((_R, 128), jnp.float32)


def _enc_body(xp_ref, pep_ref, w0b, b0b, w1b, b1b, w0ab, w0bb, ulo_ref,
              uhi_ref):
    h = jnp.maximum(
        jnp.dot(xp_ref[...], w0b[...], preferred_element_type=jnp.float32)
        + b0b[...], 0.0)
    h = jnp.dot(h, w1b[...], preferred_element_type=jnp.float32) + b1b[...]
    u = (jnp.dot(h, w0ab[...], preferred_element_type=jnp.float32)
         + jnp.dot(pep_ref[...], w0bb[...], preferred_element_type=jnp.float32))
    ulo_ref[...] = u[:, :128]
    uhi_ref[...] = u[:, 128:]


def _tc_encoder(xp, pep, w0b, b0b, w1b, b1b, w0ab, w0bb):
    return pl.pallas_call(
        _enc_body,
        grid=(_GRID,),
        in_specs=[
            pl.BlockSpec((_RB, 8 * 37), lambda i: (i, 0)),
            pl.BlockSpec((_RB, 8 * 20), lambda i: (i, 0)),
            _full((8 * 37, 256)), _full((1, 256)), _full((256, 256)),
            _full((1, 256)), _full((256, 256)), _full((8 * 20, 256)),
        ],
        out_specs=[pl.BlockSpec((_RB, 128), lambda i: (i, 0)),
                   pl.BlockSpec((_RB, 128), lambda i: (i, 0))],
        out_shape=[_packed_struct(), _packed_struct()],
    )(xp, pep, w0b, b0b, w1b, b1b, w0ab, w0bb)


def _mid_body(zlo_ref, zhi_ref, b0b, w1b, b1b, w0nb, ulo_ref, uhi_ref):
    zp = jnp.concatenate([zlo_ref[...], zhi_ref[...]], axis=1)
    t = jnp.maximum(zp + b0b[...], 0.0)
    h = jnp.maximum(
        jnp.dot(t, w1b[...], preferred_element_type=jnp.float32) + b1b[...],
        0.0)
    u = jnp.dot(h, w0nb[...], preferred_element_type=jnp.float32)
    ulo_ref[...] = u[:, :128]
    uhi_ref[...] = u[:, 128:]


def _tc_mid(z_lo, z_hi, b0b, w1b, b1b, w0nb):
    return pl.pallas_call(
        _mid_body,
        grid=(_GRID,),
        in_specs=[
            pl.BlockSpec((_RB, 128), lambda i: (i, 0)),
            pl.BlockSpec((_RB, 128), lambda i: (i, 0)),
            _full((1, 256)), _full((256, 256)), _full((1, 256)),
            _full((256, 256)),
        ],
        out_specs=[pl.BlockSpec((_RB, 128), lambda i: (i, 0)),
                   pl.BlockSpec((_RB, 128), lambda i: (i, 0))],
        out_shape=[_packed_struct(), _packed_struct()],
    )(z_lo, z_hi, b0b, w1b, b1b, w0nb)


def _fin_body(zlo_ref, zhi_ref, b8_ref, b0b, w1b, b1b, ow0b, ob0, ow1, ob1,
              out_ref, acc):
    i = pl.program_id(0)

    @pl.when(i == 0)
    def _():
        acc[...] = jnp.zeros_like(acc)

    zp = jnp.concatenate([zlo_ref[...], zhi_ref[...]], axis=1)
    t = jnp.maximum(zp + b0b[...], 0.0)
    h = jnp.maximum(
        jnp.dot(t, w1b[...], preferred_element_type=jnp.float32) + b1b[...],
        0.0)
    vp = jnp.dot(h, ow0b[...], preferred_element_type=jnp.float32)
    for n in range(8):
        bv = b8_ref[0, n, :]
        onehot = (lax.broadcasted_iota(jnp.int32, (_G, _RB), 0)
                  == bv[None, :]).astype(jnp.float32)
        vn = jnp.concatenate([vp[:, n * 16:(n + 1) * 16],
                              vp[:, 128 + n * 16:128 + (n + 1) * 16]], axis=1)
        acc[...] += jnp.dot(onehot, vn, preferred_element_type=jnp.float32)

    @pl.when(i == _GRID - 1)
    def _():
        p = jnp.maximum(acc[...] + ob0[...], 0.0)
        out_ref[...] = (jnp.dot(p, ow1[...], preferred_element_type=jnp.float32)
                        + ob1[...])


def _tc_final(z_lo, z_hi, b8, b0b, w1b, b1b, ow0b, ob0, ow1, ob1):
    return pl.pallas_call(
        _fin_body,
        grid=(_GRID,),
        in_specs=[
            pl.BlockSpec((_RB, 128), lambda i: (i, 0)),
            pl.BlockSpec((_RB, 128), lambda i: (i, 0)),
            pl.BlockSpec((1, 8, _RB), lambda i: (i, 0, 0)),
            _full((1, 256)), _full((256, 256)), _full((1, 256)),
            _full((256, 256)), _full((1, _D)), _full((_D, 2)), _full((1, 2)),
        ],
        out_specs=pl.BlockSpec((_G, 2), lambda i: (0, 0)),
        out_shape=jax.ShapeDtypeStruct((_G, 2), jnp.float32),
        scratch_shapes=[pltpu.VMEM((_G, _D), jnp.float32)],
    )(z_lo, z_hi, b8, b0b, w1b, b1b, ow0b, ob0, ow1, ob1)


def kernel(x, node_pe, edge_index, batch,
           enc_W0, enc_b0, enc_W1, enc_b1,
           gin0_W0, gin0_b0, gin0_W1, gin0_b1,
           gin1_W0, gin1_b0, gin1_W1, gin1_b1,
           gin2_W0, gin2_b0, gin2_W1, gin2_b1,
           out_W0, out_b0, out_W1, out_b1):
    src_idx = edge_index[0].reshape(_ROWS_MAIN, _IDX_W)
    dst_idx = edge_index[1].reshape(_ROWS_MAIN, _IDX_W)
    src_pad = jnp.zeros((_ROWS_PAD, _IDX_W), jnp.int32)
    dst_pad = jnp.full((_ROWS_PAD, _IDX_W), _N, jnp.int32)

    xp = jnp.pad(x, ((0, _N_PAD - _N), (0, 0))).reshape(_R, 8 * 37)
    pep = jnp.pad(node_pe, ((0, _N_PAD - _N), (0, 0))).reshape(_R, 8 * 20)
    b8 = jnp.pad(batch, (0, _N_PAD - _N),
                 constant_values=_G).reshape(_GRID, _RB, 8).transpose(0, 2, 1)

    _sc_agg = _build_sc_agg()
    packed = lambda a: a.reshape(_R, 128)
    flat = lambda a: a.reshape(_N_PAD, _H)

    u_lo, u_hi = _tc_encoder(
        xp, pep, _wbig_in(enc_W0, 37), _bbig(enc_b0), _wbig(enc_W1),
        _bbig(enc_b1), _wbig(gin0_W0[:_D]), _wbig_in(gin0_W0[_D:], 20))
    z_lo, z_hi = _sc_agg(flat(u_lo), flat(u_hi), src_idx, dst_idx,
                         src_pad, dst_pad)
    u_lo, u_hi = _tc_mid(packed(z_lo), packed(z_hi), _bbig(gin0_b0),
                         _wbig(gin0_W1), _bbig(gin0_b1), _wbig(gin1_W0))
    z_lo, z_hi = _sc_agg(flat(u_lo), flat(u_hi), src_idx, dst_idx,
                         src_pad, dst_pad)
    u_lo, u_hi = _tc_mid(packed(z_lo), packed(z_hi), _bbig(gin1_b0),
                         _wbig(gin1_W1), _bbig(gin1_b1), _wbig(gin2_W0))
    z_lo, z_hi = _sc_agg(flat(u_lo), flat(u_hi), src_idx, dst_idx,
                         src_pad, dst_pad)
    return _tc_final(packed(z_lo), packed(z_hi), b8, _bbig(gin2_b0),
                     _wbig(gin2_W1), _bbig(gin2_b1), _wbig(out_W0),
                     out_b0.reshape(1, -1), out_W1, out_b1.reshape(1, -1))
